# baseline (speedup 1.0000x reference)
import functools
import math

import jax
import jax.numpy as jnp
from jax.experimental import pallas as pl
from jax.experimental.pallas import tpu as pltpu
from jax.experimental.pallas import tpu_sc as plsc

N = 10000
NB = 100
HC = 128
NH = 8
HD = HC // NH
NRBF = 32
CUTOFF = 5.0
K = 16
MAXZ = 100
NLAYERS = 3
LSH = 8

TB = 128
NTB = 80
NP = NTB * TB
CC = 496
NCH = (NP + CC - 1) // CC
NCC = NCH * CC
TE = 32
GE = NP // TE
TN = 128
GN = NP // TN
EP = NP * K

NW = 32
EPW = EP // NW
CH = 128
NCHUNK = EPW // CH

_BETA = (2.0 / NRBF * (1.0 - math.exp(-CUTOFF))) ** -2
_NEGBIG = -3e38


def _silu(x):
    return x * jax.nn.sigmoid(x)


def _coscut(d):
    t = d * (math.pi / CUTOFF) - (math.pi / 2.0)
    t2 = t * t
    s = t * (1.0 + t2 * (-1.0 / 6 + t2 * (1.0 / 120 + t2 * (-1.0 / 5040
            + t2 * (1.0 / 362880 + t2 * (-1.0 / 39916800))))))
    return 0.5 * (1.0 - s) * (d < CUTOFF).astype(jnp.float32)


def _build_body(cand_ref, ptab_ref, bounds_ref, src_ref, val_ref):
    i = pl.program_id(0)
    blk = ptab_ref[...]
    sqd = blk[:, 3:4]
    bd = blk[:, 4:5]
    a8 = jnp.where(jax.lax.broadcasted_iota(jnp.int32, (TB, 8), 1) < 3,
                   blk[:, :8], 0.0).astype(jnp.bfloat16)
    lo = bounds_ref[i, 0]
    hi = bounds_ref[i, 1]
    jc0 = lo // CC
    nch = (hi + CC - 1) // CC - jc0
    lane_c = jax.lax.broadcasted_iota(jnp.int32, (1, CC), 1)
    neg = jnp.float32(-jnp.inf)

    def extract16(v, ix):
        w = v.shape[1]
        lane = jax.lax.broadcasted_iota(jnp.int32, (TB, w), 1)
        nv = []
        ni = []
        for _ in range(K):
            m = jnp.max(v, axis=1, keepdims=True)
            hit = v == m
            fp = jnp.min(jnp.where(hit, lane, w), axis=1, keepdims=True)
            h1 = lane == fp
            gi = jnp.sum(jnp.where(h1, ix, 0), axis=1, keepdims=True)
            nv.append(m)
            ni.append(gi)
            v = jnp.where(h1, neg, v)
        return jnp.concatenate(nv, axis=1), jnp.concatenate(ni, axis=1)

    def body(t, carry):
        topv, topi = carry
        jc = jc0 + t
        ch = cand_ref[jc]
        sqc = ch[3:4, :]
        bc = ch[4:5, :]
        b8 = jnp.where(jax.lax.broadcasted_iota(jnp.int32, (8, CC), 0) < 3,
                       ch, 0.0).astype(jnp.bfloat16)
        dot = jnp.dot(a8, b8, preferred_element_type=jnp.float32)
        d2 = sqd + sqc - 2.0 * dot
        d = jnp.sqrt(jnp.maximum(d2, 0.0))
        ok = (bd == bc) & (d < CUTOFF)
        score = jnp.where(ok, -d, neg)
        cidx = jnp.broadcast_to(jc * CC + lane_c, (TB, CC))
        half = CC // 2
        av, ai = extract16(score[:, :half], cidx[:, :half])
        bv, bi = extract16(score[:, half:], cidx[:, half:])
        cat_v = jnp.concatenate([topv, av, bv], axis=1)
        cat_i = jnp.concatenate([topi, ai, bi], axis=1)
        return extract16(cat_v, cat_i)

    topv0 = jnp.full((TB, K), neg, jnp.float32)
    topi0 = jnp.zeros((TB, K), jnp.int32)
    topv, topi = jax.lax.fori_loop(0, nch, body, (topv0, topi0))
    rows = i * TB + jax.lax.broadcasted_iota(jnp.int32, (TB, K), 0)
    fin = topv > _NEGBIG
    src_ref[...] = jnp.where(fin, topi, rows)
    val_ref[...] = topv


def _run_build(cand, ptab, bounds):
    return pl.pallas_call(
        _build_body,
        grid=(NTB,),
        in_specs=[
            pl.BlockSpec((NCH, 8, CC), lambda i: (0, 0, 0)),
            pl.BlockSpec((TB, 128), lambda i: (i, 0)),
            pl.BlockSpec(memory_space=pltpu.SMEM),
        ],
        out_specs=[
            pl.BlockSpec((TB, K), lambda i: (i, 0)),
            pl.BlockSpec((TB, K), lambda i: (i, 0)),
        ],
        out_shape=[
            jax.ShapeDtypeStruct((NP, K), jnp.int32),
            jax.ShapeDtypeStruct((NP, K), jnp.float32),
        ],
    )(cand, ptab, bounds)


def _emb_body(z_ref, embp_ref, nbrp_ref, x0_ref, xn_ref):
    zt = z_ref[...]
    oh = (zt == jax.lax.broadcasted_iota(jnp.int32, (1, 128), 1)).astype(jnp.float32)
    x0_ref[...] = jnp.dot(oh, embp_ref[...], preferred_element_type=jnp.float32)
    xn_ref[...] = jnp.dot(oh, nbrp_ref[...], preferred_element_type=jnp.float32)


def _run_emb(zp, embp, nbrp):
    return pl.pallas_call(
        _emb_body,
        grid=(GN,),
        in_specs=[
            pl.BlockSpec((TN, 1), lambda i: (i, 0)),
            pl.BlockSpec((128, HC), lambda i: (0, 0)),
            pl.BlockSpec((128, HC), lambda i: (0, 0)),
        ],
        out_specs=[
            pl.BlockSpec((TN, HC), lambda i: (i, 0)),
            pl.BlockSpec((TN, HC), lambda i: (i, 0)),
        ],
        out_shape=[
            jax.ShapeDtypeStruct((NP, HC), jnp.float32),
            jax.ShapeDtypeStruct((NP, HC), jnp.float32),
        ],
    )(zp, embp, nbrp)


def _t1_body(g0_ref, ptab_ref, x0_ref, srcm_ref, valm_ref,
             ndw_ref, ndb_ref, ncw1_ref, ncw2_ref, ncb_ref, means_ref,
             x_ref, rbf_ref, dij_ref, aux_ref):
    i = pl.program_id(0)
    ps = g0_ref[0].reshape(TE, K, 128)
    pd = ptab_ref[...]
    evx = ps[:, :, 0:1] - pd[:, None, 0:1]
    evy = ps[:, :, 1:2] - pd[:, None, 1:2]
    evz = ps[:, :, 2:3] - pd[:, None, 2:3]
    src = srcm_ref[...]
    rows = i * TE + jax.lax.broadcasted_iota(jnp.int32, (TE, K), 0)
    nsf3 = (src != rows).astype(jnp.float32)[:, :, None]
    em3 = (valm_ref[...] > _NEGBIG).astype(jnp.float32)[:, :, None]
    ns = nsf3 > 0.5
    sqd = evx * evx + evy * evy + evz * evz
    safe = jnp.sqrt(jnp.where(ns, sqd, 1.0))
    r = jnp.where(ns, safe, 0.0)
    ccut = _coscut(r)
    means = means_ref[...][None]
    rbf3 = ccut * jnp.exp(-_BETA * (jnp.exp(-r) - means) ** 2)
    safe_div = jnp.where(sqd > 0.0, safe, 1.0)
    evxn = jnp.where(ns, evx / safe_div, evx)
    evyn = jnp.where(ns, evy / safe_div, evy)
    evzn = jnp.where(ns, evz / safe_div, evz)
    s3 = math.sqrt(3.0)
    dij3 = jnp.concatenate([
        evxn, evyn, evzn,
        s3 * evxn * evzn,
        s3 * evxn * evyn,
        evyn * evyn - 0.5 * (evxn * evxn + evzn * evzn),
        s3 * evyn * evzn,
        (s3 / 2.0) * (evzn * evzn - evxn * evxn),
    ], axis=2)
    dij_ref[...] = dij3.reshape(TE * K, 8)
    rbf2 = rbf3.reshape(TE * K, NRBF)
    rbf_ref[...] = rbf2
    wt3 = (jnp.dot(rbf2, ndw_ref[...], preferred_element_type=jnp.float32)
           + ndb_ref[...]).reshape(TE, K, HC) * ccut
    ns_em = nsf3 * em3
    msg = g0_ref[1].reshape(TE, K, HC) * wt3 * ns_em
    agg = msg.sum(axis=1)
    x_ref[...] = (jnp.dot(x0_ref[...], ncw1_ref[...], preferred_element_type=jnp.float32)
                  + jnp.dot(agg, ncw2_ref[...], preferred_element_type=jnp.float32)
                  + ncb_ref[...])
    aux_ref[...] = jnp.concatenate([
        em3, ns_em, ccut, r, jnp.zeros((TE, K, 4), jnp.float32),
    ], axis=2).reshape(TE * K, 8)


def _run_t1(g0, ptab, x0, srcm, valm, ndw, ndb, ncw1, ncw2, ncb, means):
    full = lambda r, c: pl.BlockSpec((r, c), lambda i: (0, 0))
    return pl.pallas_call(
        _t1_body,
        grid=(GE,),
        in_specs=[
            pl.BlockSpec((2, TE * K, 128), lambda i: (0, i, 0)),
            pl.BlockSpec((TE, 128), lambda i: (i, 0)),
            pl.BlockSpec((TE, HC), lambda i: (i, 0)),
            pl.BlockSpec((TE, K), lambda i: (i, 0)),
            pl.BlockSpec((TE, K), lambda i: (i, 0)),
            full(NRBF, HC), full(1, HC), full(HC, HC), full(HC, HC), full(1, HC),
            full(1, NRBF),
        ],
        out_specs=[
            pl.BlockSpec((TE, HC), lambda i: (i, 0)),
            pl.BlockSpec((TE * K, NRBF), lambda i: (i, 0)),
            pl.BlockSpec((TE * K, 8), lambda i: (i, 0)),
            pl.BlockSpec((TE * K, 8), lambda i: (i, 0)),
        ],
        out_shape=[
            jax.ShapeDtypeStruct((NP, HC), jnp.float32),
            jax.ShapeDtypeStruct((EP, NRBF), jnp.float32),
            jax.ShapeDtypeStruct((EP, 8), jnp.float32),
            jax.ShapeDtypeStruct((EP, 8), jnp.float32),
        ],
    )(g0, ptab, x0, srcm, valm, ndw, ndb, ncw1, ncw2, ncb, means)


def _t4_body(mode, x_ref, *rest):
    first = mode == "first"
    last = mode == "last"
    if first:
        (lng_ref, lnb_ref, qw_ref, qb_ref, kw_ref, kb_ref, vw_ref, vb_ref,
         q_ref, cat_ref) = rest
    elif last:
        (vec_ref, lng_ref, lnb_ref, vln_ref, qw_ref, qb_ref, kw_ref, kb_ref,
         vw_ref, vb_ref, vecw_ref, q_ref, cat_ref, vd_ref, vec3_ref) = rest
    else:
        (vec_ref, lng_ref, lnb_ref, vln_ref, qw_ref, qb_ref, kw_ref, kb_ref,
         vw_ref, vb_ref, vecw_ref, wtw_ref,
         q_ref, cat_ref, vd_ref, vec3_ref, wtv_ref) = rest
    x = x_ref[...]
    mu = jnp.mean(x, axis=-1, keepdims=True)
    var = jnp.mean((x - mu) ** 2, axis=-1, keepdims=True)
    xln = (x - mu) / jnp.sqrt(var + 1e-5) * lng_ref[...] + lnb_ref[...]
    dot = lambda a, w: jnp.dot(a, w, preferred_element_type=jnp.float32)
    q_ref[...] = dot(xln, qw_ref[...]) + qb_ref[...]
    if first:
        cat_ref[0] = x
        cat_ref[1] = dot(xln, kw_ref[...]) + kb_ref[...]
        cat_ref[2] = dot(xln, vw_ref[...]) + vb_ref[...]
        return
    cat_ref[LSH // 2] = dot(xln, kw_ref[...]) + kb_ref[...]
    cat_ref[LSH // 2 + 1] = dot(xln, vw_ref[...]) + vb_ref[...]
    vln = vln_ref[...]
    acc = jnp.zeros((TN, HC), jnp.float32)
    bc = jax.lax.bitcast_convert_type
    ulo = None
    for m in range(LSH):
        vs = vec_ref[m] * vln
        u = bc(vs.astype(jnp.bfloat16).astype(jnp.float32), jnp.uint32)
        if m % 2 == 0:
            ulo = u
        else:
            cat_ref[m // 2] = bc(u | (ulo >> 16), jnp.float32)
        vp = dot(vs, vecw_ref[...])
        acc = acc + vp[:, :HC] * vp[:, HC:2 * HC]
        vec3_ref[m] = vp[:, 2 * HC:]
        if not last:
            wtv_ref[m] = dot(vs, wtw_ref[...])
    vd_ref[...] = acc


def _run_t4(mode, x, vec, lp):
    first = mode == "first"
    last = mode == "last"
    full = lambda r, c: pl.BlockSpec((r, c), lambda i: (0, 0))
    nblk = pl.BlockSpec((TN, HC), lambda i: (i, 0))
    vblk = pl.BlockSpec((LSH, TN, HC), lambda i: (0, i, 0))
    nsec = 3 if first else LSH // 2 + 2
    cblk = pl.BlockSpec((nsec, TN, HC), lambda i: (0, i, 0))
    nshape = jax.ShapeDtypeStruct((NP, HC), jnp.float32)
    vshape = jax.ShapeDtypeStruct((LSH, NP, HC), jnp.float32)
    cshape = jax.ShapeDtypeStruct((nsec, NP, HC), jnp.float32)
    wspecs = [full(1, HC), full(1, HC),
              full(HC, HC), full(1, HC), full(HC, HC), full(1, HC),
              full(HC, HC), full(1, HC)]
    wargs = [lp["ln_g"].reshape(1, HC), lp["ln_b"].reshape(1, HC),
             lp["qW"], lp["qb"].reshape(1, HC), lp["kW"], lp["kb"].reshape(1, HC),
             lp["vW"], lp["vb"].reshape(1, HC)]
    if first:
        in_specs = [nblk] + wspecs
        args = [x] + wargs
        out_specs = [nblk, cblk]
        out_shape = [nshape, cshape]
    else:
        in_specs = ([nblk, vblk, wspecs[0], wspecs[1], full(1, HC)]
                    + wspecs[2:] + [full(HC, 3 * HC)])
        args = ([x, vec, wargs[0], wargs[1], lp["vln_w"].reshape(1, HC)]
                + wargs[2:] + [lp["vecW"]])
        out_specs = [nblk, cblk, nblk, vblk]
        out_shape = [nshape, cshape, nshape, vshape]
        if not last:
            in_specs.append(full(HC, HC))
            args.append(lp["wtW"])
            out_specs.append(vblk)
            out_shape.append(vshape)
    return pl.pallas_call(
        functools.partial(_t4_body, mode),
        grid=(GN,),
        in_specs=in_specs,
        out_specs=out_specs,
        out_shape=out_shape,
    )(*args)


def _t5_body(mode, x_ref, q_ref, *rest):
    first = mode == "first"
    last = mode == "last"
    if first:
        (rbf_ref, eew_ref, eeb_ref, gath_ref, dij_ref, aux_ref,
         dkw_ref, dkb_ref, dvw_ref, dvb_ref, sw_ref, sb_ref, ow_ref, ob_ref,
         xn_ref, vecn_ref, fn_ref) = rest
    elif last:
        (vd_ref, vec_ref, vec3_ref, f_ref, gath_ref, dij_ref, aux_ref,
         dkw_ref, dkb_ref, dvw_ref, dvb_ref, sw_ref, sb_ref, ow_ref, ob_ref,
         ong_ref, onb_ref, von_ref, xn_ref, vecn_ref) = rest
    else:
        (vd_ref, vec_ref, vec3_ref, wtv_ref, f_ref, gath_ref, dij_ref, aux_ref,
         dkw_ref, dkb_ref, dvw_ref, dvb_ref, sw_ref, sb_ref, ow_ref, ob_ref,
         fw_ref, fb_ref, wsw_ref, xn_ref, vecn_ref, fn_ref) = rest
    dot = lambda a, w: jnp.dot(a, w, preferred_element_type=jnp.float32)
    if first:
        proj = dot(rbf_ref[...], eew_ref[...]) + eeb_ref[...]
        xsum = (x_ref[...][:, None, :]
                + gath_ref[0].reshape(TE, K, HC)).reshape(TE * K, HC)
        f = xsum * proj
    else:
        f = f_ref[...]
    ki = 1 if first else LSH // 2
    dk = _silu(dot(f, dkw_ref[...]) + dkb_ref[...])
    dv = _silu(dot(f, dvw_ref[...]) + dvb_ref[...])
    q3 = q_ref[...][:, None, :]
    ks3 = gath_ref[ki].reshape(TE, K, HC)
    pre = (q3 * ks3).reshape(TE * K, HC) * dk
    m1 = (jax.lax.broadcasted_iota(jnp.int32, (HC, NH), 0) // HD
          == jax.lax.broadcasted_iota(jnp.int32, (HC, NH), 1)).astype(jnp.float32)
    heads = dot(pre, m1)
    aux = aux_ref[...]
    em = aux[:, 0:1]
    ccut = aux[:, 2:3]
    ah = _silu(heads) * (ccut * em)
    m2 = (jax.lax.broadcasted_iota(jnp.int32, (NH, HC), 0)
          == jax.lax.broadcasted_iota(jnp.int32, (NH, HC), 1) // HD).astype(jnp.float32)
    attn = dot(ah, m2)
    vj = gath_ref[ki + 1] * dv * attn
    s = _silu(dot(vj, sw_ref[...]) + sb_ref[...])
    s1 = s[:, :HC] * em
    s2 = s[:, HC:] * em
    sel = (jax.lax.broadcasted_iota(jnp.int32, (TE, TE * K), 0)
           == jax.lax.broadcasted_iota(jnp.int32, (TE, TE * K), 1) // K
           ).astype(jnp.float32)
    xa = dot(sel, vj)
    o = dot(xa, ow_ref[...]) + ob_ref[...]
    o1 = o[:, :HC]
    o2 = o[:, HC:2 * HC]
    o3 = o[:, 2 * HC:]
    dij = dij_ref[...]
    if first:
        xn_ref[...] = x_ref[...] + o3
        for m in range(LSH):
            dm = dij[:, m:m + 1]
            vecn_ref[m] = dot(sel, s2 * dm)
        fn_ref[...] = f
        return
    xnew = x_ref[...] + vd_ref[...] * o2 + o3
    if not last:
        sab = jnp.zeros((TE * K, HC), jnp.float32)
        p1 = jnp.zeros((TE * K, HC), jnp.float32)
        p2p = jnp.zeros((TE * K, HC), jnp.float32)
        dd = jnp.zeros((TE * K, 1), jnp.float32)
    bc = jax.lax.bitcast_convert_type
    for m in range(LSH):
        dm = dij[:, m:m + 1]
        u = bc(gath_ref[m // 2], jnp.uint32)
        if m % 2 == 0:
            vm = bc(u << 16, jnp.float32)
        else:
            vm = bc(u & jnp.uint32(0xFFFF0000), jnp.float32)
        veca = dot(sel, vm * s1 + s2 * dm)
        vn = vec_ref[m] + vec3_ref[m] * o1 + veca
        if last:
            vecn_ref[:, m, :] = vn * von_ref[...]
        else:
            vecn_ref[m] = vn
            a_m = jnp.broadcast_to(wtv_ref[m][:, None, :], (TE, K, HC)).reshape(TE * K, HC)
            b_m = dot(vm, wsw_ref[...])
            sab = sab + a_m * b_m
            p1 = p1 + a_m * dm
            p2p = p2p + b_m * dm
            dd = dd + dm * dm
    if last:
        mu = jnp.mean(xnew, axis=-1, keepdims=True)
        var = jnp.mean((xnew - mu) ** 2, axis=-1, keepdims=True)
        xn_ref[...] = (xnew - mu) / jnp.sqrt(var + 1e-5) * ong_ref[...] + onb_ref[...]
    else:
        xn_ref[...] = xnew
        wdot = sab - p1 * p2p * (2.0 - dd)
        df = _silu(dot(f, fw_ref[...]) + fb_ref[...]) * wdot
        fn_ref[...] = f + df


def _run_t5(mode, x, q, vd, vec, vec3, wtv, f, rbf, gath, dij, aux, lp, params):
    first = mode == "first"
    last = mode == "last"
    full = lambda r, c: pl.BlockSpec((r, c), lambda i: (0, 0))
    nblk = pl.BlockSpec((TE, HC), lambda i: (i, 0))
    vblk = pl.BlockSpec((LSH, TE, HC), lambda i: (0, i, 0))
    eblk = pl.BlockSpec((TE * K, HC), lambda i: (i, 0))
    e8blk = pl.BlockSpec((TE * K, 8), lambda i: (i, 0))
    nsec = 3 if first else LSH // 2 + 2
    gblk = pl.BlockSpec((nsec, TE * K, HC), lambda i: (0, i, 0))
    wspecs = [full(HC, HC), full(1, HC), full(HC, HC), full(1, HC),
              full(HC, 2 * HC), full(1, 2 * HC), full(HC, 3 * HC), full(1, 3 * HC)]
    wargs = [lp["dkW"], lp["dkb"].reshape(1, HC), lp["dvW"], lp["dvb"].reshape(1, HC),
             lp["sW"], lp["sb"].reshape(1, 2 * HC), lp["oW"], lp["ob"].reshape(1, 3 * HC)]
    in_specs = [nblk, nblk]
    args = [x, q]
    if first:
        in_specs += [pl.BlockSpec((TE * K, NRBF), lambda i: (i, 0)),
                     full(NRBF, HC), full(1, HC)]
        args += [rbf, params["ee_W"], params["ee_b"].reshape(1, HC)]
    else:
        in_specs += [nblk, vblk, vblk]
        args += [vd, vec, vec3]
        if not last:
            in_specs.append(vblk)
            args.append(wtv)
        in_specs.append(eblk)
        args.append(f)
    in_specs += [gblk, e8blk, e8blk] + wspecs
    args += [gath, dij, aux] + wargs
    vnshape = jax.ShapeDtypeStruct((LSH, NP, HC), jnp.float32)
    if first:
        out_specs = [nblk, vblk, eblk]
        out_shape = [jax.ShapeDtypeStruct((NP, HC), jnp.float32), vnshape,
                     jax.ShapeDtypeStruct((EP, HC), jnp.float32)]
    elif last:
        in_specs += [full(1, HC), full(1, HC), full(1, HC)]
        args += [params["on_g"].reshape(1, HC), params["on_b"].reshape(1, HC),
                 params["von_w"].reshape(1, HC)]
        out_specs = [nblk, pl.BlockSpec((TE, LSH, HC), lambda i: (i, 0, 0))]
        out_shape = [jax.ShapeDtypeStruct((NP, HC), jnp.float32),
                     jax.ShapeDtypeStruct((NP, LSH, HC), jnp.float32)]
    else:
        in_specs += [full(HC, HC), full(1, HC), full(HC, HC)]
        args += [lp["fW"], lp["fb"].reshape(1, HC), lp["wsW"]]
        out_specs = [nblk, vblk, eblk]
        out_shape = [jax.ShapeDtypeStruct((NP, HC), jnp.float32), vnshape,
                     jax.ShapeDtypeStruct((EP, HC), jnp.float32)]
    return pl.pallas_call(
        functools.partial(_t5_body, mode),
        grid=(GE,),
        in_specs=in_specs,
        out_specs=out_specs,
        out_shape=out_shape,
    )(*args)


NBUF = 4
NCHT = EP // CH


def _gather_cat(table, nsec, idx):
    cpw = -(-NCHT // (NW // nsec))
    cpw = -(-cpw // NBUF) * NBUF
    rpw = NW // nsec
    idx2d = (idx[None, :]
             + (jnp.arange(nsec, dtype=jnp.int32) * NP)[:, None]).reshape(-1)
    out_type = jax.ShapeDtypeStruct((nsec * EP, HC), jnp.float32)
    mesh = plsc.VectorSubcoreMesh(core_axis_name="c", subcore_axis_name="s",
                                  num_cores=2, num_subcores=16)
    scratch = ([pltpu.VMEM((cpw * CH,), jnp.int32)]
               + [pltpu.VMEM((CH, HC), jnp.float32) for _ in range(NBUF)]
               + [pltpu.SemaphoreType.DMA for _ in range(2 * NBUF)])

    def body(tab, idx_hbm, out, idxv, *scr):
        bufs = scr[:NBUF]
        sg = scr[NBUF:2 * NBUF]
        sw = scr[2 * NBUF:]
        wid = jax.lax.axis_index("s") * 2 + jax.lax.axis_index("c")
        sec = wid // rpw
        k = wid % rpw
        c0 = jnp.minimum(k * cpw, NCHT - cpw)
        obase = sec * EP + c0 * CH

        @pl.when(sec < nsec)
        def _():
            pltpu.sync_copy(idx_hbm.at[pl.ds(obase, cpw * CH)], idxv)

            def service(rel2, b2):
                pltpu.make_async_copy(
                    tab.at[idxv.at[pl.ds(rel2 * CH, CH)]],
                    bufs[b2], sg[b2]).wait()
                pltpu.async_copy(bufs[b2],
                                 out.at[pl.ds(obase + rel2 * CH, CH)],
                                 sw[b2])

            def step(j, b):
                rel = NBUF * j + b

                @pl.when(j >= 1)
                def _():
                    pltpu.make_async_copy(
                        bufs[b], out.at[pl.ds(obase, CH)], sw[b]).wait()
                pltpu.async_copy(tab.at[idxv.at[pl.ds(rel * CH, CH)]],
                                 bufs[b], sg[b])
                if b >= 2:
                    service(rel - 2, b - 2)
                else:
                    @pl.when(j >= 1)
                    def _():
                        service(rel - 2, b + 2)

            def outer(j, carry):
                for b in range(NBUF):
                    step(j, b)
                return carry

            jax.lax.fori_loop(0, cpw // NBUF, outer, 0)
            service(cpw - 2, NBUF - 2)
            service(cpw - 1, NBUF - 1)
            for s in range(NBUF):
                pltpu.make_async_copy(
                    bufs[s], out.at[pl.ds(obase, CH)], sw[s]).wait()

    fn = pl.kernel(body, out_type=out_type, mesh=mesh, scratch_types=scratch)
    return fn(table, idx2d)


def kernel(pos, z, batch, params):
    f32 = jnp.float32
    npad = NP - N
    px = pos[:, 0]
    py = pos[:, 1]
    pz = pos[:, 2]
    sq = px * px + py * py + pz * pz
    batchf = batch.astype(f32)

    def padv(v, fill):
        return jnp.concatenate([v, jnp.full((npad,), fill, v.dtype)])

    pxp = padv(px, 0.0)
    pyp = padv(py, 0.0)
    pzp = padv(pz, 0.0)
    sqp = padv(sq, 0.0)
    bfp = padv(batchf, float(2 ** 30))
    ptab = jnp.concatenate(
        [jnp.stack([pxp, pyp, pzp, sqp, bfp], axis=1),
         jnp.zeros((NP, 123), f32)], axis=1)

    cpad = NCC - NP
    crows = jnp.stack([
        jnp.concatenate([pxp, jnp.zeros((cpad,), f32)]),
        jnp.concatenate([pyp, jnp.zeros((cpad,), f32)]),
        jnp.concatenate([pzp, jnp.zeros((cpad,), f32)]),
        jnp.concatenate([sqp, jnp.zeros((cpad,), f32)]),
        jnp.concatenate([bfp, jnp.full((cpad,), float(2 ** 31), f32)]),
        jnp.zeros((NCC,), f32), jnp.zeros((NCC,), f32), jnp.zeros((NCC,), f32),
    ], axis=0)
    cand = jnp.transpose(crows.reshape(8, NCH, CC), (1, 0, 2))

    batch_pad = jnp.concatenate([batch.astype(jnp.int32),
                                 jnp.full((npad,), 2 ** 30, jnp.int32)])
    t0 = jnp.arange(NTB, dtype=jnp.int32) * TB
    blo = batch_pad[t0]
    bhi = batch_pad[t0 + TB - 1]
    lo = jnp.searchsorted(batch_pad, blo, side="left").astype(jnp.int32)
    hi = jnp.searchsorted(batch_pad, bhi, side="right").astype(jnp.int32)
    bounds = jnp.stack([lo, hi], axis=1)

    srcm, valm = _run_build(cand, ptab, bounds)
    idx = srcm.reshape(-1)

    zp = padv(z.astype(jnp.int32), 0).reshape(NP, 1)
    embp = jnp.zeros((128, HC), f32).at[:MAXZ].set(params["emb"])
    nbrp = jnp.zeros((128, HC), f32).at[:MAXZ].set(params["nbr_emb"])
    x0, xn = _run_emb(zp, embp, nbrp)

    g0 = _gather_cat(jnp.concatenate([ptab, xn], axis=0), 2, idx).reshape(2, EP, 128)

    means = jnp.linspace(math.exp(-CUTOFF), 1.0, NRBF).astype(f32).reshape(1, NRBF)
    x, rbf, dij, aux = _run_t1(
        g0, ptab, x0, srcm, valm,
        params["nd_W"], params["nd_b"].reshape(1, HC),
        params["nc_W"][:HC], params["nc_W"][HC:], params["nc_b"].reshape(1, HC),
        means)

    vec = f = vd = vec3 = wtv = None
    for li in range(NLAYERS):
        mode = "first" if li == 0 else ("last" if li == NLAYERS - 1 else "mid")
        lp = params["layers"][li]
        outs = _run_t4(mode, x, vec, lp)
        if mode == "first":
            q, cat = outs
            nsec = 3
        elif mode == "last":
            q, cat, vd, vec3 = outs
            nsec = LSH // 2 + 2
        else:
            q, cat, vd, vec3, wtv = outs
            nsec = LSH // 2 + 2
        gath = _gather_cat(cat.reshape(nsec * NP, HC), nsec, idx)
        gath = gath.reshape(nsec, EP, HC)
        outs = _run_t5(mode, x, q, vd, vec, vec3, wtv, f, rbf, gath,
                       dij, aux, lp, params)
        if mode == "last":
            x, vec_out = outs
        else:
            x, vec, f = outs

    return x[:N], vec_out[:N]


# speedup vs baseline: 1.1134x; 1.1134x over previous
import functools
import math

import jax
import jax.numpy as jnp
from jax.experimental import pallas as pl
from jax.experimental.pallas import tpu as pltpu
from jax.experimental.pallas import tpu_sc as plsc

N = 10000
NB = 100
HC = 128
NH = 8
HD = HC // NH
NRBF = 32
CUTOFF = 5.0
K = 16
MAXZ = 100
NLAYERS = 3
LSH = 8

TB = 128
NTB = 80
NP = NTB * TB
CC = 496
NCH = (NP + CC - 1) // CC
NCC = NCH * CC
TE = 32
GE = NP // TE
TN = 128
GN = NP // TN
EP = NP * K

NW = 32
EPW = EP // NW
CH = 128
NCHUNK = EPW // CH

_BETA = (2.0 / NRBF * (1.0 - math.exp(-CUTOFF))) ** -2
_NEGBIG = -3e38


def _silu(x):
    return x * jax.nn.sigmoid(x)


def _coscut(d):
    t = d * (math.pi / CUTOFF) - (math.pi / 2.0)
    t2 = t * t
    s = t * (1.0 + t2 * (-1.0 / 6 + t2 * (1.0 / 120 + t2 * (-1.0 / 5040
            + t2 * (1.0 / 362880 + t2 * (-1.0 / 39916800))))))
    return 0.5 * (1.0 - s) * (d < CUTOFF).astype(jnp.float32)


def _build_body(cand_ref, ptab_ref, bounds_ref, src_ref, val_ref):
    i = pl.program_id(0)
    blk = ptab_ref[...]
    sqd = blk[:, 3:4]
    bd = blk[:, 4:5]
    a8 = jnp.where(jax.lax.broadcasted_iota(jnp.int32, (TB, 8), 1) < 3,
                   blk[:, :8], 0.0).astype(jnp.bfloat16)
    lo = bounds_ref[i, 0]
    hi = bounds_ref[i, 1]
    jc0 = lo // CC
    nch = (hi + CC - 1) // CC - jc0
    lane_c = jax.lax.broadcasted_iota(jnp.int32, (1, CC), 1)
    neg = jnp.float32(-jnp.inf)

    def extract16(v, ix):
        w = v.shape[1]
        lane = jax.lax.broadcasted_iota(jnp.int32, (TB, w), 1)
        nv = []
        ni = []
        for _ in range(K):
            m = jnp.max(v, axis=1, keepdims=True)
            hit = v == m
            fp = jnp.min(jnp.where(hit, lane, w), axis=1, keepdims=True)
            h1 = lane == fp
            gi = jnp.sum(jnp.where(h1, ix, 0), axis=1, keepdims=True)
            nv.append(m)
            ni.append(gi)
            v = jnp.where(h1, neg, v)
        return jnp.concatenate(nv, axis=1), jnp.concatenate(ni, axis=1)

    def body(t, carry):
        topv, topi = carry
        jc = jc0 + t
        ch = cand_ref[jc]
        sqc = ch[3:4, :]
        bc = ch[4:5, :]
        b8 = jnp.where(jax.lax.broadcasted_iota(jnp.int32, (8, CC), 0) < 3,
                       ch, 0.0).astype(jnp.bfloat16)
        dot = jnp.dot(a8, b8, preferred_element_type=jnp.float32)
        d2 = sqd + sqc - 2.0 * dot
        d = jnp.sqrt(jnp.maximum(d2, 0.0))
        ok = (bd == bc) & (d < CUTOFF)
        score = jnp.where(ok, -d, neg)
        cidx = jnp.broadcast_to(jc * CC + lane_c, (TB, CC))
        cat_v = jnp.concatenate([topv, score], axis=1)
        cat_i = jnp.concatenate([topi, cidx], axis=1)
        return extract16(cat_v, cat_i)

    topv0 = jnp.full((TB, K), neg, jnp.float32)
    topi0 = jnp.zeros((TB, K), jnp.int32)
    topv, topi = jax.lax.fori_loop(0, nch, body, (topv0, topi0))
    rows = i * TB + jax.lax.broadcasted_iota(jnp.int32, (TB, K), 0)
    fin = topv > _NEGBIG
    src_ref[...] = jnp.where(fin, topi, rows)
    val_ref[...] = topv


def _run_build(cand, ptab, bounds):
    return pl.pallas_call(
        _build_body,
        grid=(NTB,),
        in_specs=[
            pl.BlockSpec((NCH, 8, CC), lambda i: (0, 0, 0)),
            pl.BlockSpec((TB, 128), lambda i: (i, 0)),
            pl.BlockSpec(memory_space=pltpu.SMEM),
        ],
        out_specs=[
            pl.BlockSpec((TB, K), lambda i: (i, 0)),
            pl.BlockSpec((TB, K), lambda i: (i, 0)),
        ],
        out_shape=[
            jax.ShapeDtypeStruct((NP, K), jnp.int32),
            jax.ShapeDtypeStruct((NP, K), jnp.float32),
        ],
    )(cand, ptab, bounds)


def _emb_body(z_ref, embp_ref, nbrp_ref, x0_ref, xn_ref):
    zt = z_ref[...]
    oh = (zt == jax.lax.broadcasted_iota(jnp.int32, (1, 128), 1)).astype(jnp.float32)
    x0_ref[...] = jnp.dot(oh, embp_ref[...], preferred_element_type=jnp.float32)
    xn_ref[...] = jnp.dot(oh, nbrp_ref[...], preferred_element_type=jnp.float32)


def _run_emb(zp, embp, nbrp):
    return pl.pallas_call(
        _emb_body,
        grid=(GN,),
        in_specs=[
            pl.BlockSpec((TN, 1), lambda i: (i, 0)),
            pl.BlockSpec((128, HC), lambda i: (0, 0)),
            pl.BlockSpec((128, HC), lambda i: (0, 0)),
        ],
        out_specs=[
            pl.BlockSpec((TN, HC), lambda i: (i, 0)),
            pl.BlockSpec((TN, HC), lambda i: (i, 0)),
        ],
        out_shape=[
            jax.ShapeDtypeStruct((NP, HC), jnp.float32),
            jax.ShapeDtypeStruct((NP, HC), jnp.float32),
        ],
    )(zp, embp, nbrp)


def _t1_body(g0_ref, ptab_ref, x0_ref, srcm_ref, valm_ref,
             ndw_ref, ndb_ref, ncw1_ref, ncw2_ref, ncb_ref, means_ref,
             x_ref, rbf_ref, dij_ref, aux_ref):
    i = pl.program_id(0)
    ps = g0_ref[0].reshape(TE, K, 128)
    pd = ptab_ref[...]
    evx = ps[:, :, 0:1] - pd[:, None, 0:1]
    evy = ps[:, :, 1:2] - pd[:, None, 1:2]
    evz = ps[:, :, 2:3] - pd[:, None, 2:3]
    src = srcm_ref[...]
    rows = i * TE + jax.lax.broadcasted_iota(jnp.int32, (TE, K), 0)
    nsf3 = (src != rows).astype(jnp.float32)[:, :, None]
    em3 = (valm_ref[...] > _NEGBIG).astype(jnp.float32)[:, :, None]
    ns = nsf3 > 0.5
    sqd = evx * evx + evy * evy + evz * evz
    safe = jnp.sqrt(jnp.where(ns, sqd, 1.0))
    r = jnp.where(ns, safe, 0.0)
    ccut = _coscut(r)
    means = means_ref[...][None]
    rbf3 = ccut * jnp.exp(-_BETA * (jnp.exp(-r) - means) ** 2)
    safe_div = jnp.where(sqd > 0.0, safe, 1.0)
    evxn = jnp.where(ns, evx / safe_div, evx)
    evyn = jnp.where(ns, evy / safe_div, evy)
    evzn = jnp.where(ns, evz / safe_div, evz)
    s3 = math.sqrt(3.0)
    dij3 = jnp.concatenate([
        evxn, evyn, evzn,
        s3 * evxn * evzn,
        s3 * evxn * evyn,
        evyn * evyn - 0.5 * (evxn * evxn + evzn * evzn),
        s3 * evyn * evzn,
        (s3 / 2.0) * (evzn * evzn - evxn * evxn),
    ], axis=2)
    dij_ref[...] = dij3.reshape(TE * K, 8)
    rbf2 = rbf3.reshape(TE * K, NRBF)
    rbf_ref[...] = rbf2
    wt3 = (jnp.dot(rbf2, ndw_ref[...], preferred_element_type=jnp.float32)
           + ndb_ref[...]).reshape(TE, K, HC) * ccut
    ns_em = nsf3 * em3
    msg = g0_ref[1].reshape(TE, K, HC) * wt3 * ns_em
    agg = msg.sum(axis=1)
    x_ref[...] = (jnp.dot(x0_ref[...], ncw1_ref[...], preferred_element_type=jnp.float32)
                  + jnp.dot(agg, ncw2_ref[...], preferred_element_type=jnp.float32)
                  + ncb_ref[...])
    aux_ref[...] = jnp.concatenate([
        em3, ns_em, ccut, r, jnp.zeros((TE, K, 4), jnp.float32),
    ], axis=2).reshape(TE * K, 8)


def _run_t1(g0, ptab, x0, srcm, valm, ndw, ndb, ncw1, ncw2, ncb, means):
    full = lambda r, c: pl.BlockSpec((r, c), lambda i: (0, 0))
    return pl.pallas_call(
        _t1_body,
        grid=(GE,),
        in_specs=[
            pl.BlockSpec((2, TE * K, 128), lambda i: (0, i, 0)),
            pl.BlockSpec((TE, 128), lambda i: (i, 0)),
            pl.BlockSpec((TE, HC), lambda i: (i, 0)),
            pl.BlockSpec((TE, K), lambda i: (i, 0)),
            pl.BlockSpec((TE, K), lambda i: (i, 0)),
            full(NRBF, HC), full(1, HC), full(HC, HC), full(HC, HC), full(1, HC),
            full(1, NRBF),
        ],
        out_specs=[
            pl.BlockSpec((TE, HC), lambda i: (i, 0)),
            pl.BlockSpec((TE * K, NRBF), lambda i: (i, 0)),
            pl.BlockSpec((TE * K, 8), lambda i: (i, 0)),
            pl.BlockSpec((TE * K, 8), lambda i: (i, 0)),
        ],
        out_shape=[
            jax.ShapeDtypeStruct((NP, HC), jnp.float32),
            jax.ShapeDtypeStruct((EP, NRBF), jnp.float32),
            jax.ShapeDtypeStruct((EP, 8), jnp.float32),
            jax.ShapeDtypeStruct((EP, 8), jnp.float32),
        ],
    )(g0, ptab, x0, srcm, valm, ndw, ndb, ncw1, ncw2, ncb, means)


def _t4_body(mode, x_ref, *rest):
    first = mode == "first"
    last = mode == "last"
    if first:
        (lng_ref, lnb_ref, qw_ref, qb_ref, kw_ref, kb_ref, vw_ref, vb_ref,
         q_ref, cat_ref) = rest
    elif last:
        (vec_ref, lng_ref, lnb_ref, vln_ref, qw_ref, qb_ref, kw_ref, kb_ref,
         vw_ref, vb_ref, vecw_ref, q_ref, cat_ref, vd_ref, vec3_ref) = rest
    else:
        (vec_ref, lng_ref, lnb_ref, vln_ref, qw_ref, qb_ref, kw_ref, kb_ref,
         vw_ref, vb_ref, vecw_ref, wtw_ref,
         q_ref, cat_ref, vd_ref, vec3_ref, wtv_ref) = rest
    x = x_ref[...]
    mu = jnp.mean(x, axis=-1, keepdims=True)
    var = jnp.mean((x - mu) ** 2, axis=-1, keepdims=True)
    xln = (x - mu) / jnp.sqrt(var + 1e-5) * lng_ref[...] + lnb_ref[...]
    dot = lambda a, w: jnp.dot(a, w, preferred_element_type=jnp.float32)
    q_ref[...] = dot(xln, qw_ref[...]) + qb_ref[...]
    if first:
        cat_ref[0] = x
        cat_ref[1] = dot(xln, kw_ref[...]) + kb_ref[...]
        cat_ref[2] = dot(xln, vw_ref[...]) + vb_ref[...]
        return
    cat_ref[LSH // 2] = dot(xln, kw_ref[...]) + kb_ref[...]
    cat_ref[LSH // 2 + 1] = dot(xln, vw_ref[...]) + vb_ref[...]
    vln = vln_ref[...]
    acc = jnp.zeros((TN, HC), jnp.float32)
    bc = jax.lax.bitcast_convert_type
    ulo = None
    for m in range(LSH):
        vs = vec_ref[m] * vln
        u = bc(vs.astype(jnp.bfloat16).astype(jnp.float32), jnp.uint32)
        if m % 2 == 0:
            ulo = u
        else:
            cat_ref[m // 2] = bc(u | (ulo >> 16), jnp.float32)
        vp = dot(vs, vecw_ref[...])
        acc = acc + vp[:, :HC] * vp[:, HC:2 * HC]
        vec3_ref[m] = vp[:, 2 * HC:]
        if not last:
            wtv_ref[m] = dot(vs, wtw_ref[...])
    vd_ref[...] = acc


def _run_t4(mode, x, vec, lp):
    first = mode == "first"
    last = mode == "last"
    full = lambda r, c: pl.BlockSpec((r, c), lambda i: (0, 0))
    nblk = pl.BlockSpec((TN, HC), lambda i: (i, 0))
    vblk = pl.BlockSpec((LSH, TN, HC), lambda i: (0, i, 0))
    nsec = 3 if first else LSH // 2 + 2
    cblk = pl.BlockSpec((nsec, TN, HC), lambda i: (0, i, 0))
    nshape = jax.ShapeDtypeStruct((NP, HC), jnp.float32)
    vshape = jax.ShapeDtypeStruct((LSH, NP, HC), jnp.float32)
    cshape = jax.ShapeDtypeStruct((nsec, NP, HC), jnp.float32)
    wspecs = [full(1, HC), full(1, HC),
              full(HC, HC), full(1, HC), full(HC, HC), full(1, HC),
              full(HC, HC), full(1, HC)]
    wargs = [lp["ln_g"].reshape(1, HC), lp["ln_b"].reshape(1, HC),
             lp["qW"], lp["qb"].reshape(1, HC), lp["kW"], lp["kb"].reshape(1, HC),
             lp["vW"], lp["vb"].reshape(1, HC)]
    if first:
        in_specs = [nblk] + wspecs
        args = [x] + wargs
        out_specs = [nblk, cblk]
        out_shape = [nshape, cshape]
    else:
        in_specs = ([nblk, vblk, wspecs[0], wspecs[1], full(1, HC)]
                    + wspecs[2:] + [full(HC, 3 * HC)])
        args = ([x, vec, wargs[0], wargs[1], lp["vln_w"].reshape(1, HC)]
                + wargs[2:] + [lp["vecW"]])
        out_specs = [nblk, cblk, nblk, vblk]
        out_shape = [nshape, cshape, nshape, vshape]
        if not last:
            in_specs.append(full(HC, HC))
            args.append(lp["wtW"])
            out_specs.append(vblk)
            out_shape.append(vshape)
    return pl.pallas_call(
        functools.partial(_t4_body, mode),
        grid=(GN,),
        in_specs=in_specs,
        out_specs=out_specs,
        out_shape=out_shape,
    )(*args)


def _t5_body(mode, x_ref, q_ref, *rest):
    first = mode == "first"
    last = mode == "last"
    if first:
        (rbf_ref, eew_ref, eeb_ref, gath_ref, dij_ref, aux_ref,
         dkw_ref, dkb_ref, dvw_ref, dvb_ref, sw_ref, sb_ref, ow_ref, ob_ref,
         xn_ref, vecn_ref, fn_ref) = rest
    elif last:
        (vd_ref, vec_ref, vec3_ref, f_ref, gath_ref, dij_ref, aux_ref,
         dkw_ref, dkb_ref, dvw_ref, dvb_ref, sw_ref, sb_ref, ow_ref, ob_ref,
         ong_ref, onb_ref, von_ref, xn_ref, vecn_ref) = rest
    else:
        (vd_ref, vec_ref, vec3_ref, wtv_ref, f_ref, gath_ref, dij_ref, aux_ref,
         dkw_ref, dkb_ref, dvw_ref, dvb_ref, sw_ref, sb_ref, ow_ref, ob_ref,
         fw_ref, fb_ref, wsw_ref, xn_ref, vecn_ref, fn_ref) = rest
    dot = lambda a, w: jnp.dot(a, w, preferred_element_type=jnp.float32)
    if first:
        proj = dot(rbf_ref[...], eew_ref[...]) + eeb_ref[...]
        xsum = (x_ref[...][:, None, :]
                + gath_ref[0].reshape(TE, K, HC)).reshape(TE * K, HC)
        f = xsum * proj
    else:
        f = f_ref[...]
    ki = 1 if first else LSH // 2
    dk = _silu(dot(f, dkw_ref[...]) + dkb_ref[...])
    dv = _silu(dot(f, dvw_ref[...]) + dvb_ref[...])
    q3 = q_ref[...][:, None, :]
    ks3 = gath_ref[ki].reshape(TE, K, HC)
    pre = (q3 * ks3).reshape(TE * K, HC) * dk
    m1 = (jax.lax.broadcasted_iota(jnp.int32, (HC, NH), 0) // HD
          == jax.lax.broadcasted_iota(jnp.int32, (HC, NH), 1)).astype(jnp.float32)
    heads = dot(pre, m1)
    aux = aux_ref[...]
    em = aux[:, 0:1]
    ccut = aux[:, 2:3]
    ah = _silu(heads) * (ccut * em)
    m2 = (jax.lax.broadcasted_iota(jnp.int32, (NH, HC), 0)
          == jax.lax.broadcasted_iota(jnp.int32, (NH, HC), 1) // HD).astype(jnp.float32)
    attn = dot(ah, m2)
    vj = gath_ref[ki + 1] * dv * attn
    s = _silu(dot(vj, sw_ref[...]) + sb_ref[...])
    s1 = s[:, :HC] * em
    s2 = s[:, HC:] * em
    sel = (jax.lax.broadcasted_iota(jnp.int32, (TE, TE * K), 0)
           == jax.lax.broadcasted_iota(jnp.int32, (TE, TE * K), 1) // K
           ).astype(jnp.float32)
    xa = dot(sel, vj)
    o = dot(xa, ow_ref[...]) + ob_ref[...]
    o1 = o[:, :HC]
    o2 = o[:, HC:2 * HC]
    o3 = o[:, 2 * HC:]
    dij = dij_ref[...]
    if first:
        xn_ref[...] = x_ref[...] + o3
        for m in range(LSH):
            dm = dij[:, m:m + 1]
            vecn_ref[m] = dot(sel, s2 * dm)
        fn_ref[...] = f
        return
    xnew = x_ref[...] + vd_ref[...] * o2 + o3
    if not last:
        sab = jnp.zeros((TE * K, HC), jnp.float32)
        p1 = jnp.zeros((TE * K, HC), jnp.float32)
        p2p = jnp.zeros((TE * K, HC), jnp.float32)
        dd = jnp.zeros((TE * K, 1), jnp.float32)
    bc = jax.lax.bitcast_convert_type
    for m in range(LSH):
        dm = dij[:, m:m + 1]
        u = bc(gath_ref[m // 2], jnp.uint32)
        if m % 2 == 0:
            vm = bc(u << 16, jnp.float32)
        else:
            vm = bc(u & jnp.uint32(0xFFFF0000), jnp.float32)
        veca = dot(sel, vm * s1 + s2 * dm)
        vn = vec_ref[m] + vec3_ref[m] * o1 + veca
        if last:
            vecn_ref[:, m, :] = vn * von_ref[...]
        else:
            vecn_ref[m] = vn
            a_m = jnp.broadcast_to(wtv_ref[m][:, None, :], (TE, K, HC)).reshape(TE * K, HC)
            b_m = dot(vm, wsw_ref[...])
            sab = sab + a_m * b_m
            p1 = p1 + a_m * dm
            p2p = p2p + b_m * dm
            dd = dd + dm * dm
    if last:
        mu = jnp.mean(xnew, axis=-1, keepdims=True)
        var = jnp.mean((xnew - mu) ** 2, axis=-1, keepdims=True)
        xn_ref[...] = (xnew - mu) / jnp.sqrt(var + 1e-5) * ong_ref[...] + onb_ref[...]
    else:
        xn_ref[...] = xnew
        wdot = sab - p1 * p2p * (2.0 - dd)
        df = _silu(dot(f, fw_ref[...]) + fb_ref[...]) * wdot
        fn_ref[...] = f + df


def _run_t5(mode, x, q, vd, vec, vec3, wtv, f, rbf, gath, dij, aux, lp, params):
    first = mode == "first"
    last = mode == "last"
    full = lambda r, c: pl.BlockSpec((r, c), lambda i: (0, 0))
    nblk = pl.BlockSpec((TE, HC), lambda i: (i, 0))
    vblk = pl.BlockSpec((LSH, TE, HC), lambda i: (0, i, 0))
    eblk = pl.BlockSpec((TE * K, HC), lambda i: (i, 0))
    e8blk = pl.BlockSpec((TE * K, 8), lambda i: (i, 0))
    nsec = 3 if first else LSH // 2 + 2
    gblk = pl.BlockSpec((nsec, TE * K, HC), lambda i: (0, i, 0))
    wspecs = [full(HC, HC), full(1, HC), full(HC, HC), full(1, HC),
              full(HC, 2 * HC), full(1, 2 * HC), full(HC, 3 * HC), full(1, 3 * HC)]
    wargs = [lp["dkW"], lp["dkb"].reshape(1, HC), lp["dvW"], lp["dvb"].reshape(1, HC),
             lp["sW"], lp["sb"].reshape(1, 2 * HC), lp["oW"], lp["ob"].reshape(1, 3 * HC)]
    in_specs = [nblk, nblk]
    args = [x, q]
    if first:
        in_specs += [pl.BlockSpec((TE * K, NRBF), lambda i: (i, 0)),
                     full(NRBF, HC), full(1, HC)]
        args += [rbf, params["ee_W"], params["ee_b"].reshape(1, HC)]
    else:
        in_specs += [nblk, vblk, vblk]
        args += [vd, vec, vec3]
        if not last:
            in_specs.append(vblk)
            args.append(wtv)
        in_specs.append(eblk)
        args.append(f)
    in_specs += [gblk, e8blk, e8blk] + wspecs
    args += [gath, dij, aux] + wargs
    vnshape = jax.ShapeDtypeStruct((LSH, NP, HC), jnp.float32)
    if first:
        out_specs = [nblk, vblk, eblk]
        out_shape = [jax.ShapeDtypeStruct((NP, HC), jnp.float32), vnshape,
                     jax.ShapeDtypeStruct((EP, HC), jnp.float32)]
    elif last:
        in_specs += [full(1, HC), full(1, HC), full(1, HC)]
        args += [params["on_g"].reshape(1, HC), params["on_b"].reshape(1, HC),
                 params["von_w"].reshape(1, HC)]
        out_specs = [nblk, pl.BlockSpec((TE, LSH, HC), lambda i: (i, 0, 0))]
        out_shape = [jax.ShapeDtypeStruct((NP, HC), jnp.float32),
                     jax.ShapeDtypeStruct((NP, LSH, HC), jnp.float32)]
    else:
        in_specs += [full(HC, HC), full(1, HC), full(HC, HC)]
        args += [lp["fW"], lp["fb"].reshape(1, HC), lp["wsW"]]
        out_specs = [nblk, vblk, eblk]
        out_shape = [jax.ShapeDtypeStruct((NP, HC), jnp.float32), vnshape,
                     jax.ShapeDtypeStruct((EP, HC), jnp.float32)]
    return pl.pallas_call(
        functools.partial(_t5_body, mode),
        grid=(GE,),
        in_specs=in_specs,
        out_specs=out_specs,
        out_shape=out_shape,
    )(*args)


NBUF = 4
NCHT = EP // CH


def _gather_cat(table, nsec, idx):
    cpw = -(-NCHT // (NW // nsec))
    cpw = -(-cpw // NBUF) * NBUF
    rpw = NW // nsec
    idx2d = (idx[None, :]
             + (jnp.arange(nsec, dtype=jnp.int32) * NP)[:, None]).reshape(-1)
    out_type = jax.ShapeDtypeStruct((nsec * EP, HC), jnp.float32)
    mesh = plsc.VectorSubcoreMesh(core_axis_name="c", subcore_axis_name="s",
                                  num_cores=2, num_subcores=16)
    scratch = ([pltpu.VMEM((cpw * CH,), jnp.int32)]
               + [pltpu.VMEM((CH, HC), jnp.float32) for _ in range(NBUF)]
               + [pltpu.SemaphoreType.DMA for _ in range(2 * NBUF)])

    def body(tab, idx_hbm, out, idxv, *scr):
        bufs = scr[:NBUF]
        sg = scr[NBUF:2 * NBUF]
        sw = scr[2 * NBUF:]
        wid = jax.lax.axis_index("s") * 2 + jax.lax.axis_index("c")
        sec = wid // rpw
        k = wid % rpw
        c0 = jnp.minimum(k * cpw, NCHT - cpw)
        obase = sec * EP + c0 * CH

        @pl.when(sec < nsec)
        def _():
            pltpu.sync_copy(idx_hbm.at[pl.ds(obase, cpw * CH)], idxv)

            def service(rel2, b2):
                pltpu.make_async_copy(
                    tab.at[idxv.at[pl.ds(rel2 * CH, CH)]],
                    bufs[b2], sg[b2]).wait()
                pltpu.async_copy(bufs[b2],
                                 out.at[pl.ds(obase + rel2 * CH, CH)],
                                 sw[b2])

            def step(j, b):
                rel = NBUF * j + b

                @pl.when(j >= 1)
                def _():
                    pltpu.make_async_copy(
                        bufs[b], out.at[pl.ds(obase, CH)], sw[b]).wait()
                pltpu.async_copy(tab.at[idxv.at[pl.ds(rel * CH, CH)]],
                                 bufs[b], sg[b])
                if b >= 2:
                    service(rel - 2, b - 2)
                else:
                    @pl.when(j >= 1)
                    def _():
                        service(rel - 2, b + 2)

            def outer(j, carry):
                for b in range(NBUF):
                    step(j, b)
                return carry

            jax.lax.fori_loop(0, cpw // NBUF, outer, 0)
            service(cpw - 2, NBUF - 2)
            service(cpw - 1, NBUF - 1)
            for s in range(NBUF):
                pltpu.make_async_copy(
                    bufs[s], out.at[pl.ds(obase, CH)], sw[s]).wait()

    fn = pl.kernel(body, out_type=out_type, mesh=mesh, scratch_types=scratch)
    return fn(table, idx2d)


def kernel(pos, z, batch, params):
    f32 = jnp.float32
    npad = NP - N
    px = pos[:, 0]
    py = pos[:, 1]
    pz = pos[:, 2]
    sq = px * px + py * py + pz * pz
    batchf = batch.astype(f32)

    def padv(v, fill):
        return jnp.concatenate([v, jnp.full((npad,), fill, v.dtype)])

    pxp = padv(px, 0.0)
    pyp = padv(py, 0.0)
    pzp = padv(pz, 0.0)
    sqp = padv(sq, 0.0)
    bfp = padv(batchf, float(2 ** 30))
    ptab = jnp.concatenate(
        [jnp.stack([pxp, pyp, pzp, sqp, bfp], axis=1),
         jnp.zeros((NP, 123), f32)], axis=1)

    cpad = NCC - NP
    crows = jnp.stack([
        jnp.concatenate([pxp, jnp.zeros((cpad,), f32)]),
        jnp.concatenate([pyp, jnp.zeros((cpad,), f32)]),
        jnp.concatenate([pzp, jnp.zeros((cpad,), f32)]),
        jnp.concatenate([sqp, jnp.zeros((cpad,), f32)]),
        jnp.concatenate([bfp, jnp.full((cpad,), float(2 ** 31), f32)]),
        jnp.zeros((NCC,), f32), jnp.zeros((NCC,), f32), jnp.zeros((NCC,), f32),
    ], axis=0)
    cand = jnp.transpose(crows.reshape(8, NCH, CC), (1, 0, 2))

    batch_pad = jnp.concatenate([batch.astype(jnp.int32),
                                 jnp.full((npad,), 2 ** 30, jnp.int32)])
    t0 = jnp.arange(NTB, dtype=jnp.int32) * TB
    blo = batch_pad[t0]
    bhi = batch_pad[t0 + TB - 1]
    lo = jnp.searchsorted(batch_pad, blo, side="left").astype(jnp.int32)
    hi = jnp.searchsorted(batch_pad, bhi, side="right").astype(jnp.int32)
    bounds = jnp.stack([lo, hi], axis=1)

    srcm, valm = _run_build(cand, ptab, bounds)
    idx = srcm.reshape(-1)

    zp = padv(z.astype(jnp.int32), 0).reshape(NP, 1)
    embp = jnp.zeros((128, HC), f32).at[:MAXZ].set(params["emb"])
    nbrp = jnp.zeros((128, HC), f32).at[:MAXZ].set(params["nbr_emb"])
    x0, xn = _run_emb(zp, embp, nbrp)

    g0 = _gather_cat(jnp.concatenate([ptab, xn], axis=0), 2, idx).reshape(2, EP, 128)

    means = jnp.linspace(math.exp(-CUTOFF), 1.0, NRBF).astype(f32).reshape(1, NRBF)
    x, rbf, dij, aux = _run_t1(
        g0, ptab, x0, srcm, valm,
        params["nd_W"], params["nd_b"].reshape(1, HC),
        params["nc_W"][:HC], params["nc_W"][HC:], params["nc_b"].reshape(1, HC),
        means)

    vec = f = vd = vec3 = wtv = None
    for li in range(NLAYERS):
        mode = "first" if li == 0 else ("last" if li == NLAYERS - 1 else "mid")
        lp = params["layers"][li]
        outs = _run_t4(mode, x, vec, lp)
        if mode == "first":
            q, cat = outs
            nsec = 3
        elif mode == "last":
            q, cat, vd, vec3 = outs
            nsec = LSH // 2 + 2
        else:
            q, cat, vd, vec3, wtv = outs
            nsec = LSH // 2 + 2
        gath = _gather_cat(cat.reshape(nsec * NP, HC), nsec, idx)
        gath = gath.reshape(nsec, EP, HC)
        outs = _run_t5(mode, x, q, vd, vec, vec3, wtv, f, rbf, gath,
                       dij, aux, lp, params)
        if mode == "last":
            x, vec_out = outs
        else:
            x, vec, f = outs

    return x[:N], vec_out[:N]


# speedup vs baseline: 1.1141x; 1.0006x over previous
import functools
import math

import jax
import jax.numpy as jnp
from jax.experimental import pallas as pl
from jax.experimental.pallas import tpu as pltpu
from jax.experimental.pallas import tpu_sc as plsc

N = 10000
NB = 100
HC = 128
NH = 8
HD = HC // NH
NRBF = 32
CUTOFF = 5.0
K = 16
MAXZ = 100
NLAYERS = 3
LSH = 8

TB = 128
NTB = 80
NP = NTB * TB
CC = 496
NCH = (NP + CC - 1) // CC
NCC = NCH * CC
TE = 32
GE = NP // TE
TN = 128
GN = NP // TN
EP = NP * K

NW = 32
EPW = EP // NW
CH = 128
NCHUNK = EPW // CH

_BETA = (2.0 / NRBF * (1.0 - math.exp(-CUTOFF))) ** -2
_NEGBIG = -3e38


def _silu(x):
    return x * jax.nn.sigmoid(x)


def _coscut(d):
    t = d * (math.pi / CUTOFF) - (math.pi / 2.0)
    t2 = t * t
    s = t * (1.0 + t2 * (-1.0 / 6 + t2 * (1.0 / 120 + t2 * (-1.0 / 5040
            + t2 * (1.0 / 362880 + t2 * (-1.0 / 39916800))))))
    return 0.5 * (1.0 - s) * (d < CUTOFF).astype(jnp.float32)


def _build_body(cand_ref, ptab_ref, bounds_ref, src_ref, val_ref):
    i = pl.program_id(0)
    blk = ptab_ref[...]
    sqd = blk[:, 3:4]
    bd = blk[:, 4:5]
    a8 = jnp.where(jax.lax.broadcasted_iota(jnp.int32, (TB, 8), 1) < 3,
                   blk[:, :8], 0.0).astype(jnp.bfloat16)
    lo = bounds_ref[i, 0]
    hi = bounds_ref[i, 1]
    jc0 = lo // CC
    nch = (hi + CC - 1) // CC - jc0
    lane_c = jax.lax.broadcasted_iota(jnp.int32, (1, CC), 1)
    neg = jnp.float32(-jnp.inf)

    def extract16(v, ix):
        w = v.shape[1]
        lane = jax.lax.broadcasted_iota(jnp.int32, (TB, w), 1)
        nv = []
        ni = []
        for _ in range(K):
            m = jnp.max(v, axis=1, keepdims=True)
            hit = v == m
            fp = jnp.min(jnp.where(hit, lane, w), axis=1, keepdims=True)
            h1 = lane == fp
            gi = jnp.sum(jnp.where(h1, ix, 0), axis=1, keepdims=True)
            nv.append(m)
            ni.append(gi)
            v = jnp.where(h1, neg, v)
        return jnp.concatenate(nv, axis=1), jnp.concatenate(ni, axis=1)

    def body(t, carry):
        topv, topi = carry
        jc = jc0 + t
        ch = cand_ref[jc]
        sqc = ch[3:4, :]
        bc = ch[4:5, :]
        b8 = jnp.where(jax.lax.broadcasted_iota(jnp.int32, (8, CC), 0) < 3,
                       ch, 0.0).astype(jnp.bfloat16)
        dot = jnp.dot(a8, b8, preferred_element_type=jnp.float32)
        d2 = sqd + sqc - 2.0 * dot
        d = jnp.sqrt(jnp.maximum(d2, 0.0))
        ok = (bd == bc) & (d < CUTOFF)
        score = jnp.where(ok, -d, neg)
        cidx = jnp.broadcast_to(jc * CC + lane_c, (TB, CC))
        cat_v = jnp.concatenate([topv, score], axis=1)
        cat_i = jnp.concatenate([topi, cidx], axis=1)
        return extract16(cat_v, cat_i)

    topv0 = jnp.full((TB, K), neg, jnp.float32)
    topi0 = jnp.zeros((TB, K), jnp.int32)
    topv, topi = jax.lax.fori_loop(0, nch, body, (topv0, topi0))
    rows = i * TB + jax.lax.broadcasted_iota(jnp.int32, (TB, K), 0)
    fin = topv > _NEGBIG
    src_ref[...] = jnp.where(fin, topi, rows)
    val_ref[...] = topv


def _run_build(cand, ptab, bounds):
    return pl.pallas_call(
        _build_body,
        grid=(NTB,),
        in_specs=[
            pl.BlockSpec((NCH, 8, CC), lambda i: (0, 0, 0)),
            pl.BlockSpec((TB, 128), lambda i: (i, 0)),
            pl.BlockSpec(memory_space=pltpu.SMEM),
        ],
        out_specs=[
            pl.BlockSpec((TB, K), lambda i: (i, 0)),
            pl.BlockSpec((TB, K), lambda i: (i, 0)),
        ],
        out_shape=[
            jax.ShapeDtypeStruct((NP, K), jnp.int32),
            jax.ShapeDtypeStruct((NP, K), jnp.float32),
        ],
    )(cand, ptab, bounds)


def _emb_body(z_ref, embp_ref, nbrp_ref, x0_ref, xn_ref):
    zt = z_ref[...]
    oh = (zt == jax.lax.broadcasted_iota(jnp.int32, (1, 128), 1)).astype(jnp.float32)
    x0_ref[...] = jnp.dot(oh, embp_ref[...], preferred_element_type=jnp.float32)
    xn_ref[...] = jnp.dot(oh, nbrp_ref[...], preferred_element_type=jnp.float32)


def _run_emb(zp, embp, nbrp):
    return pl.pallas_call(
        _emb_body,
        grid=(GN,),
        in_specs=[
            pl.BlockSpec((TN, 1), lambda i: (i, 0)),
            pl.BlockSpec((128, HC), lambda i: (0, 0)),
            pl.BlockSpec((128, HC), lambda i: (0, 0)),
        ],
        out_specs=[
            pl.BlockSpec((TN, HC), lambda i: (i, 0)),
            pl.BlockSpec((TN, HC), lambda i: (i, 0)),
        ],
        out_shape=[
            jax.ShapeDtypeStruct((NP, HC), jnp.float32),
            jax.ShapeDtypeStruct((NP, HC), jnp.float32),
        ],
    )(zp, embp, nbrp)


def _t1_body(g0_ref, ptab_ref, x0_ref, srcm_ref, valm_ref,
             ndw_ref, ndb_ref, ncw1_ref, ncw2_ref, ncb_ref, means_ref,
             x_ref, rbf_ref, dij_ref, aux_ref):
    i = pl.program_id(0)
    ps = g0_ref[0].reshape(TE, K, 128)
    pd = ptab_ref[...]
    evx = ps[:, :, 0:1] - pd[:, None, 0:1]
    evy = ps[:, :, 1:2] - pd[:, None, 1:2]
    evz = ps[:, :, 2:3] - pd[:, None, 2:3]
    src = srcm_ref[...]
    rows = i * TE + jax.lax.broadcasted_iota(jnp.int32, (TE, K), 0)
    nsf3 = (src != rows).astype(jnp.float32)[:, :, None]
    em3 = (valm_ref[...] > _NEGBIG).astype(jnp.float32)[:, :, None]
    ns = nsf3 > 0.5
    sqd = evx * evx + evy * evy + evz * evz
    safe = jnp.sqrt(jnp.where(ns, sqd, 1.0))
    r = jnp.where(ns, safe, 0.0)
    ccut = _coscut(r)
    means = means_ref[...][None]
    rbf3 = ccut * jnp.exp(-_BETA * (jnp.exp(-r) - means) ** 2)
    safe_div = jnp.where(sqd > 0.0, safe, 1.0)
    evxn = jnp.where(ns, evx / safe_div, evx)
    evyn = jnp.where(ns, evy / safe_div, evy)
    evzn = jnp.where(ns, evz / safe_div, evz)
    s3 = math.sqrt(3.0)
    dij3 = jnp.concatenate([
        evxn, evyn, evzn,
        s3 * evxn * evzn,
        s3 * evxn * evyn,
        evyn * evyn - 0.5 * (evxn * evxn + evzn * evzn),
        s3 * evyn * evzn,
        (s3 / 2.0) * (evzn * evzn - evxn * evxn),
    ], axis=2)
    dij_ref[...] = dij3.reshape(TE * K, 8)
    rbf2 = rbf3.reshape(TE * K, NRBF)
    rbf_ref[...] = rbf2
    wt3 = (jnp.dot(rbf2, ndw_ref[...], preferred_element_type=jnp.float32)
           + ndb_ref[...]).reshape(TE, K, HC) * ccut
    ns_em = nsf3 * em3
    msg = g0_ref[1].reshape(TE, K, HC) * wt3 * ns_em
    agg = msg.sum(axis=1)
    x_ref[...] = (jnp.dot(x0_ref[...], ncw1_ref[...], preferred_element_type=jnp.float32)
                  + jnp.dot(agg, ncw2_ref[...], preferred_element_type=jnp.float32)
                  + ncb_ref[...])
    aux_ref[...] = jnp.concatenate([
        em3, ns_em, ccut, r, jnp.zeros((TE, K, 4), jnp.float32),
    ], axis=2).reshape(TE * K, 8)


def _run_t1(g0, ptab, x0, srcm, valm, ndw, ndb, ncw1, ncw2, ncb, means):
    full = lambda r, c: pl.BlockSpec((r, c), lambda i: (0, 0))
    return pl.pallas_call(
        _t1_body,
        grid=(GE,),
        in_specs=[
            pl.BlockSpec((2, TE * K, 128), lambda i: (0, i, 0)),
            pl.BlockSpec((TE, 128), lambda i: (i, 0)),
            pl.BlockSpec((TE, HC), lambda i: (i, 0)),
            pl.BlockSpec((TE, K), lambda i: (i, 0)),
            pl.BlockSpec((TE, K), lambda i: (i, 0)),
            full(NRBF, HC), full(1, HC), full(HC, HC), full(HC, HC), full(1, HC),
            full(1, NRBF),
        ],
        out_specs=[
            pl.BlockSpec((TE, HC), lambda i: (i, 0)),
            pl.BlockSpec((TE * K, NRBF), lambda i: (i, 0)),
            pl.BlockSpec((TE * K, 8), lambda i: (i, 0)),
            pl.BlockSpec((TE * K, 8), lambda i: (i, 0)),
        ],
        out_shape=[
            jax.ShapeDtypeStruct((NP, HC), jnp.float32),
            jax.ShapeDtypeStruct((EP, NRBF), jnp.float32),
            jax.ShapeDtypeStruct((EP, 8), jnp.float32),
            jax.ShapeDtypeStruct((EP, 8), jnp.float32),
        ],
    )(g0, ptab, x0, srcm, valm, ndw, ndb, ncw1, ncw2, ncb, means)


def _t4_body(mode, x_ref, *rest):
    first = mode == "first"
    last = mode == "last"
    if first:
        (lng_ref, lnb_ref, qw_ref, qb_ref, kw_ref, kb_ref, vw_ref, vb_ref,
         q_ref, cat_ref) = rest
    elif last:
        (vec_ref, lng_ref, lnb_ref, vln_ref, qw_ref, qb_ref, kw_ref, kb_ref,
         vw_ref, vb_ref, vecw_ref, q_ref, cat_ref, vd_ref, vec3_ref) = rest
    else:
        (vec_ref, lng_ref, lnb_ref, vln_ref, qw_ref, qb_ref, kw_ref, kb_ref,
         vw_ref, vb_ref, vecw_ref, wtw_ref,
         q_ref, cat_ref, vd_ref, vec3_ref, wtv_ref) = rest
    x = x_ref[...]
    mu = jnp.mean(x, axis=-1, keepdims=True)
    var = jnp.mean((x - mu) ** 2, axis=-1, keepdims=True)
    xln = (x - mu) / jnp.sqrt(var + 1e-5) * lng_ref[...] + lnb_ref[...]
    dot = lambda a, w: jnp.dot(a, w, preferred_element_type=jnp.float32)
    q_ref[...] = dot(xln, qw_ref[...]) + qb_ref[...]
    if first:
        cat_ref[0] = x
        cat_ref[1] = dot(xln, kw_ref[...]) + kb_ref[...]
        cat_ref[2] = dot(xln, vw_ref[...]) + vb_ref[...]
        return
    cat_ref[LSH // 2] = dot(xln, kw_ref[...]) + kb_ref[...]
    cat_ref[LSH // 2 + 1] = dot(xln, vw_ref[...]) + vb_ref[...]
    vln = vln_ref[...]
    acc = jnp.zeros((TN, HC), jnp.float32)
    bc = jax.lax.bitcast_convert_type
    ulo = None
    for m in range(LSH):
        vs = vec_ref[m] * vln
        u = bc(vs.astype(jnp.bfloat16).astype(jnp.float32), jnp.uint32)
        if m % 2 == 0:
            ulo = u
        else:
            cat_ref[m // 2] = bc(u | (ulo >> 16), jnp.float32)
        vp = dot(vs, vecw_ref[...])
        acc = acc + vp[:, :HC] * vp[:, HC:2 * HC]
        vec3_ref[m] = vp[:, 2 * HC:]
        if not last:
            wtv_ref[m] = dot(vs, wtw_ref[...])
    vd_ref[...] = acc


def _run_t4(mode, x, vec, lp):
    first = mode == "first"
    last = mode == "last"
    full = lambda r, c: pl.BlockSpec((r, c), lambda i: (0, 0))
    nblk = pl.BlockSpec((TN, HC), lambda i: (i, 0))
    vblk = pl.BlockSpec((LSH, TN, HC), lambda i: (0, i, 0))
    nsec = 3 if first else LSH // 2 + 2
    cblk = pl.BlockSpec((nsec, TN, HC), lambda i: (0, i, 0))
    nshape = jax.ShapeDtypeStruct((NP, HC), jnp.float32)
    vshape = jax.ShapeDtypeStruct((LSH, NP, HC), jnp.float32)
    cshape = jax.ShapeDtypeStruct((nsec, NP, HC), jnp.float32)
    wspecs = [full(1, HC), full(1, HC),
              full(HC, HC), full(1, HC), full(HC, HC), full(1, HC),
              full(HC, HC), full(1, HC)]
    wargs = [lp["ln_g"].reshape(1, HC), lp["ln_b"].reshape(1, HC),
             lp["qW"], lp["qb"].reshape(1, HC), lp["kW"], lp["kb"].reshape(1, HC),
             lp["vW"], lp["vb"].reshape(1, HC)]
    if first:
        in_specs = [nblk] + wspecs
        args = [x] + wargs
        out_specs = [nblk, cblk]
        out_shape = [nshape, cshape]
    else:
        in_specs = ([nblk, vblk, wspecs[0], wspecs[1], full(1, HC)]
                    + wspecs[2:] + [full(HC, 3 * HC)])
        args = ([x, vec, wargs[0], wargs[1], lp["vln_w"].reshape(1, HC)]
                + wargs[2:] + [lp["vecW"]])
        out_specs = [nblk, cblk, nblk, vblk]
        out_shape = [nshape, cshape, nshape, vshape]
        if not last:
            in_specs.append(full(HC, HC))
            args.append(lp["wtW"])
            out_specs.append(vblk)
            out_shape.append(vshape)
    return pl.pallas_call(
        functools.partial(_t4_body, mode),
        grid=(GN,),
        in_specs=in_specs,
        out_specs=out_specs,
        out_shape=out_shape,
    )(*args)


def _t5_body(mode, x_ref, q_ref, *rest):
    first = mode == "first"
    last = mode == "last"
    if first:
        (rbf_ref, eew_ref, eeb_ref, gath_ref, dij_ref, aux_ref,
         dkw_ref, dkb_ref, dvw_ref, dvb_ref, sw_ref, sb_ref, ow_ref, ob_ref,
         xn_ref, vecn_ref, fn_ref) = rest
    elif last:
        (vd_ref, vec_ref, vec3_ref, f_ref, gath_ref, dij_ref, aux_ref,
         dkw_ref, dkb_ref, dvw_ref, dvb_ref, sw_ref, sb_ref, ow_ref, ob_ref,
         ong_ref, onb_ref, von_ref, xn_ref, vecn_ref) = rest
    else:
        (vd_ref, vec_ref, vec3_ref, wtv_ref, f_ref, gath_ref, dij_ref, aux_ref,
         dkw_ref, dkb_ref, dvw_ref, dvb_ref, sw_ref, sb_ref, ow_ref, ob_ref,
         fw_ref, fb_ref, wsw_ref, xn_ref, vecn_ref, fn_ref) = rest
    dot = lambda a, w: jnp.dot(a, w, preferred_element_type=jnp.float32)
    if first:
        proj = dot(rbf_ref[...], eew_ref[...]) + eeb_ref[...]
        xsum = (x_ref[...][:, None, :]
                + gath_ref[0].reshape(TE, K, HC)).reshape(TE * K, HC)
        f = xsum * proj
    else:
        f = f_ref[...]
    ki = 1 if first else LSH // 2
    dk = _silu(dot(f, dkw_ref[...]) + dkb_ref[...])
    dv = _silu(dot(f, dvw_ref[...]) + dvb_ref[...])
    q3 = q_ref[...][:, None, :]
    ks3 = gath_ref[ki].reshape(TE, K, HC)
    pre = (q3 * ks3).reshape(TE * K, HC) * dk
    m1 = (jax.lax.broadcasted_iota(jnp.int32, (HC, NH), 0) // HD
          == jax.lax.broadcasted_iota(jnp.int32, (HC, NH), 1)).astype(jnp.float32)
    heads = dot(pre, m1)
    aux = aux_ref[...]
    em = aux[:, 0:1]
    ccut = aux[:, 2:3]
    ah = _silu(heads) * (ccut * em)
    m2 = (jax.lax.broadcasted_iota(jnp.int32, (NH, HC), 0)
          == jax.lax.broadcasted_iota(jnp.int32, (NH, HC), 1) // HD).astype(jnp.float32)
    attn = dot(ah, m2)
    vj = gath_ref[ki + 1] * dv * attn
    s = _silu(dot(vj, sw_ref[...]) + sb_ref[...])
    s1 = s[:, :HC] * em
    s2 = s[:, HC:] * em
    sel = (jax.lax.broadcasted_iota(jnp.int32, (TE, TE * K), 0)
           == jax.lax.broadcasted_iota(jnp.int32, (TE, TE * K), 1) // K
           ).astype(jnp.float32)
    xa = dot(sel, vj)
    o = dot(xa, ow_ref[...]) + ob_ref[...]
    o1 = o[:, :HC]
    o2 = o[:, HC:2 * HC]
    o3 = o[:, 2 * HC:]
    dij = dij_ref[...]
    if first:
        xn_ref[...] = x_ref[...] + o3
        for m in range(LSH):
            dm = dij[:, m:m + 1]
            vecn_ref[m] = dot(sel, s2 * dm)
        fn_ref[...] = f
        return
    xnew = x_ref[...] + vd_ref[...] * o2 + o3
    if not last:
        sab = jnp.zeros((TE * K, HC), jnp.float32)
        p1 = jnp.zeros((TE * K, HC), jnp.float32)
        p2p = jnp.zeros((TE * K, HC), jnp.float32)
        dd = jnp.zeros((TE * K, 1), jnp.float32)
    bc = jax.lax.bitcast_convert_type
    for m in range(LSH):
        dm = dij[:, m:m + 1]
        u = bc(gath_ref[m // 2], jnp.uint32)
        if m % 2 == 0:
            vm = bc(u << 16, jnp.float32)
        else:
            vm = bc(u & jnp.uint32(0xFFFF0000), jnp.float32)
        veca = dot(sel, vm * s1 + s2 * dm)
        vn = vec_ref[m] + vec3_ref[m] * o1 + veca
        if last:
            vecn_ref[:, m, :] = vn * von_ref[...]
        else:
            vecn_ref[m] = vn
            a_m = jnp.broadcast_to(wtv_ref[m][:, None, :], (TE, K, HC)).reshape(TE * K, HC)
            b_m = dot(vm, wsw_ref[...])
            sab = sab + a_m * b_m
            p1 = p1 + a_m * dm
            p2p = p2p + b_m * dm
            dd = dd + dm * dm
    if last:
        mu = jnp.mean(xnew, axis=-1, keepdims=True)
        var = jnp.mean((xnew - mu) ** 2, axis=-1, keepdims=True)
        xn_ref[...] = (xnew - mu) / jnp.sqrt(var + 1e-5) * ong_ref[...] + onb_ref[...]
    else:
        xn_ref[...] = xnew
        wdot = sab - p1 * p2p * (2.0 - dd)
        df = _silu(dot(f, fw_ref[...]) + fb_ref[...]) * wdot
        fn_ref[...] = f + df


def _run_t5(mode, x, q, vd, vec, vec3, wtv, f, rbf, gath, dij, aux, lp, params):
    first = mode == "first"
    last = mode == "last"
    full = lambda r, c: pl.BlockSpec((r, c), lambda i: (0, 0))
    nblk = pl.BlockSpec((TE, HC), lambda i: (i, 0))
    vblk = pl.BlockSpec((LSH, TE, HC), lambda i: (0, i, 0))
    eblk = pl.BlockSpec((TE * K, HC), lambda i: (i, 0))
    e8blk = pl.BlockSpec((TE * K, 8), lambda i: (i, 0))
    nsec = 3 if first else LSH // 2 + 2
    gblk = pl.BlockSpec((nsec, TE * K, HC), lambda i: (0, i, 0))
    wspecs = [full(HC, HC), full(1, HC), full(HC, HC), full(1, HC),
              full(HC, 2 * HC), full(1, 2 * HC), full(HC, 3 * HC), full(1, 3 * HC)]
    wargs = [lp["dkW"], lp["dkb"].reshape(1, HC), lp["dvW"], lp["dvb"].reshape(1, HC),
             lp["sW"], lp["sb"].reshape(1, 2 * HC), lp["oW"], lp["ob"].reshape(1, 3 * HC)]
    in_specs = [nblk, nblk]
    args = [x, q]
    if first:
        in_specs += [pl.BlockSpec((TE * K, NRBF), lambda i: (i, 0)),
                     full(NRBF, HC), full(1, HC)]
        args += [rbf, params["ee_W"], params["ee_b"].reshape(1, HC)]
    else:
        in_specs += [nblk, vblk, vblk]
        args += [vd, vec, vec3]
        if not last:
            in_specs.append(vblk)
            args.append(wtv)
        in_specs.append(eblk)
        args.append(f)
    in_specs += [gblk, e8blk, e8blk] + wspecs
    args += [gath, dij, aux] + wargs
    vnshape = jax.ShapeDtypeStruct((LSH, NP, HC), jnp.float32)
    if first:
        out_specs = [nblk, vblk, eblk]
        out_shape = [jax.ShapeDtypeStruct((NP, HC), jnp.float32), vnshape,
                     jax.ShapeDtypeStruct((EP, HC), jnp.float32)]
    elif last:
        in_specs += [full(1, HC), full(1, HC), full(1, HC)]
        args += [params["on_g"].reshape(1, HC), params["on_b"].reshape(1, HC),
                 params["von_w"].reshape(1, HC)]
        out_specs = [nblk, pl.BlockSpec((TE, LSH, HC), lambda i: (i, 0, 0))]
        out_shape = [jax.ShapeDtypeStruct((NP, HC), jnp.float32),
                     jax.ShapeDtypeStruct((NP, LSH, HC), jnp.float32)]
    else:
        in_specs += [full(HC, HC), full(1, HC), full(HC, HC)]
        args += [lp["fW"], lp["fb"].reshape(1, HC), lp["wsW"]]
        out_specs = [nblk, vblk, eblk]
        out_shape = [jax.ShapeDtypeStruct((NP, HC), jnp.float32), vnshape,
                     jax.ShapeDtypeStruct((EP, HC), jnp.float32)]
    return pl.pallas_call(
        functools.partial(_t5_body, mode),
        grid=(GE,),
        in_specs=in_specs,
        out_specs=out_specs,
        out_shape=out_shape,
    )(*args)


NBUF = 4
NCHT = EP // CH


def _gather_cat(table, nsec, idx):
    cpw = -(-NCHT // (NW // nsec))
    cpw = -(-cpw // NBUF) * NBUF
    rpw = NW // nsec
    idx2d = (idx[None, :]
             + (jnp.arange(nsec, dtype=jnp.int32) * NP)[:, None]).reshape(-1)
    out_type = jax.ShapeDtypeStruct((nsec * EP, HC), jnp.float32)
    mesh = plsc.VectorSubcoreMesh(core_axis_name="c", subcore_axis_name="s",
                                  num_cores=2, num_subcores=16)
    scratch = ([pltpu.VMEM((cpw * CH,), jnp.int32)]
               + [pltpu.VMEM((CH, HC), jnp.float32) for _ in range(NBUF)]
               + [pltpu.SemaphoreType.DMA for _ in range(2 * NBUF)])

    def body(tab, idx_hbm, out, idxv, *scr):
        bufs = scr[:NBUF]
        sg = scr[NBUF:2 * NBUF]
        sw = scr[2 * NBUF:]
        wid = jax.lax.axis_index("s") * 2 + jax.lax.axis_index("c")
        sec = wid // rpw
        k = wid % rpw
        c0 = jnp.minimum(k * cpw, NCHT - cpw)
        obase = sec * EP + c0 * CH

        @pl.when(sec < nsec)
        def _():
            pltpu.sync_copy(idx_hbm.at[pl.ds(obase, cpw * CH)], idxv)

            def service(rel2, b2):
                pltpu.make_async_copy(
                    tab.at[idxv.at[pl.ds(rel2 * CH, CH)]],
                    bufs[b2], sg[b2]).wait()
                pltpu.async_copy(bufs[b2],
                                 out.at[pl.ds(obase + rel2 * CH, CH)],
                                 sw[b2])

            def step(j, b):
                rel = NBUF * j + b

                @pl.when(j >= 1)
                def _():
                    pltpu.make_async_copy(
                        bufs[b], out.at[pl.ds(obase, CH)], sw[b]).wait()
                pltpu.async_copy(tab.at[idxv.at[pl.ds(rel * CH, CH)]],
                                 bufs[b], sg[b])
                if b >= 3:
                    service(rel - 3, b - 3)
                else:
                    @pl.when(j >= 1)
                    def _():
                        service(rel - 3, b + 1)

            def outer(j, carry):
                for b in range(NBUF):
                    step(j, b)
                return carry

            jax.lax.fori_loop(0, cpw // NBUF, outer, 0)
            service(cpw - 3, NBUF - 3)
            service(cpw - 2, NBUF - 2)
            service(cpw - 1, NBUF - 1)
            for s in range(NBUF):
                pltpu.make_async_copy(
                    bufs[s], out.at[pl.ds(obase, CH)], sw[s]).wait()

    fn = pl.kernel(body, out_type=out_type, mesh=mesh, scratch_types=scratch)
    return fn(table, idx2d)


def kernel(pos, z, batch, params):
    f32 = jnp.float32
    npad = NP - N
    px = pos[:, 0]
    py = pos[:, 1]
    pz = pos[:, 2]
    sq = px * px + py * py + pz * pz
    batchf = batch.astype(f32)

    def padv(v, fill):
        return jnp.concatenate([v, jnp.full((npad,), fill, v.dtype)])

    pxp = padv(px, 0.0)
    pyp = padv(py, 0.0)
    pzp = padv(pz, 0.0)
    sqp = padv(sq, 0.0)
    bfp = padv(batchf, float(2 ** 30))
    ptab = jnp.concatenate(
        [jnp.stack([pxp, pyp, pzp, sqp, bfp], axis=1),
         jnp.zeros((NP, 123), f32)], axis=1)

    cpad = NCC - NP
    crows = jnp.stack([
        jnp.concatenate([pxp, jnp.zeros((cpad,), f32)]),
        jnp.concatenate([pyp, jnp.zeros((cpad,), f32)]),
        jnp.concatenate([pzp, jnp.zeros((cpad,), f32)]),
        jnp.concatenate([sqp, jnp.zeros((cpad,), f32)]),
        jnp.concatenate([bfp, jnp.full((cpad,), float(2 ** 31), f32)]),
        jnp.zeros((NCC,), f32), jnp.zeros((NCC,), f32), jnp.zeros((NCC,), f32),
    ], axis=0)
    cand = jnp.transpose(crows.reshape(8, NCH, CC), (1, 0, 2))

    batch_pad = jnp.concatenate([batch.astype(jnp.int32),
                                 jnp.full((npad,), 2 ** 30, jnp.int32)])
    t0 = jnp.arange(NTB, dtype=jnp.int32) * TB
    blo = batch_pad[t0]
    bhi = batch_pad[t0 + TB - 1]
    lo = jnp.searchsorted(batch_pad, blo, side="left").astype(jnp.int32)
    hi = jnp.searchsorted(batch_pad, bhi, side="right").astype(jnp.int32)
    bounds = jnp.stack([lo, hi], axis=1)

    srcm, valm = _run_build(cand, ptab, bounds)
    idx = srcm.reshape(-1)

    zp = padv(z.astype(jnp.int32), 0).reshape(NP, 1)
    embp = jnp.zeros((128, HC), f32).at[:MAXZ].set(params["emb"])
    nbrp = jnp.zeros((128, HC), f32).at[:MAXZ].set(params["nbr_emb"])
    x0, xn = _run_emb(zp, embp, nbrp)

    g0 = _gather_cat(jnp.concatenate([ptab, xn], axis=0), 2, idx).reshape(2, EP, 128)

    means = jnp.linspace(math.exp(-CUTOFF), 1.0, NRBF).astype(f32).reshape(1, NRBF)
    x, rbf, dij, aux = _run_t1(
        g0, ptab, x0, srcm, valm,
        params["nd_W"], params["nd_b"].reshape(1, HC),
        params["nc_W"][:HC], params["nc_W"][HC:], params["nc_b"].reshape(1, HC),
        means)

    vec = f = vd = vec3 = wtv = None
    for li in range(NLAYERS):
        mode = "first" if li == 0 else ("last" if li == NLAYERS - 1 else "mid")
        lp = params["layers"][li]
        outs = _run_t4(mode, x, vec, lp)
        if mode == "first":
            q, cat = outs
            nsec = 3
        elif mode == "last":
            q, cat, vd, vec3 = outs
            nsec = LSH // 2 + 2
        else:
            q, cat, vd, vec3, wtv = outs
            nsec = LSH // 2 + 2
        gath = _gather_cat(cat.reshape(nsec * NP, HC), nsec, idx)
        gath = gath.reshape(nsec, EP, HC)
        outs = _run_t5(mode, x, q, vd, vec, vec3, wtv, f, rbf, gath,
                       dij, aux, lp, params)
        if mode == "last":
            x, vec_out = outs
        else:
            x, vec, f = outs

    return x[:N], vec_out[:N]
